# baseline (speedup 1.0000x reference)
import jax
import jax.numpy as jnp
from jax import lax
from jax.experimental import pallas as pl
from jax.experimental.pallas import tpu as pltpu
from jax.experimental.pallas import tpu_sc as plsc

_N = 2048
_E = 65536
_DF = 128
_H = 32
_P1 = 1024
_P2 = 512
_NCLS = 10


_NC = 2
_NS = 16
_LANES = 16
_ROWS_PP = 512
_PASSES = _N // (_NC * _ROWS_PP)
_EPT = _E // _NS
_CHUNK = 128
_NCHUNK = _EPT // _CHUNK
_ZBUF = 8192
_PANEL = _ROWS_PP * _N
_SP_PER_TILE = _PANEL // _NS
_ROWS_PER_TILE = _ROWS_PP // _NS


def _adj_body(edge_hbm, a_hbm, deg_hbm, src_v, dst_v, idx_v, upd_v, didx_v,
              zero_v, spmem):
    c = lax.axis_index("c")
    s = lax.axis_index("s")
    e0 = s * _EPT

    pltpu.sync_copy(edge_hbm.at[0, pl.ds(e0, _EPT)], src_v)
    pltpu.sync_copy(edge_hbm.at[1, pl.ds(e0, _EPT)], dst_v)

    @pl.loop(0, _ZBUF, step=_LANES)
    def _(i):
        zero_v[pl.ds(i, _LANES)] = jnp.zeros((_LANES,), jnp.float32)

    @pl.loop(0, _PASSES)
    def _(p):
        row_base = c * (_PASSES * _ROWS_PP) + p * _ROWS_PP

        @pl.loop(0, _SP_PER_TILE, step=_ZBUF)
        def _(z):
            pltpu.sync_copy(zero_v, spmem.at[pl.ds(s * _SP_PER_TILE + z, _ZBUF)])

        @pl.when(s == 0)
        def _():
            pltpu.sync_copy(zero_v.at[pl.ds(0, _ROWS_PP)],
                            spmem.at[pl.ds(_PANEL, _ROWS_PP)])

        @pl.loop(0, _NCHUNK)
        def _(j):
            @pl.loop(0, _CHUNK, step=_LANES)
            def _(k):
                off = j * _CHUNK + k
                src = src_v[pl.ds(off, _LANES)]
                dst = dst_v[pl.ds(off, _LANES)]
                rel = src - row_base
                inb = (rel >= 0) & (rel < _ROWS_PP)
                row = rel & (_ROWS_PP - 1)
                idx_v[j, pl.ds(k, _LANES)] = row * _N + dst
                didx_v[j, pl.ds(k, _LANES)] = _PANEL + row
                upd_v[j, pl.ds(k, _LANES)] = jnp.where(
                    inb, jnp.float32(1.0), jnp.float32(0.0))

        plsc.subcore_barrier()

        @pl.loop(0, _NCHUNK)
        def _(j):
            pltpu.sync_copy(upd_v.at[j], spmem.at[idx_v.at[j]], add=True)

        @pl.loop(0, _NCHUNK)
        def _(j):
            pltpu.sync_copy(upd_v.at[j], spmem.at[didx_v.at[j]], add=True)

        plsc.subcore_barrier()

        @pl.loop(0, _ROWS_PER_TILE)
        def _(r):
            lr = s * _ROWS_PER_TILE + r
            pltpu.sync_copy(spmem.at[pl.ds(lr * _N, _N)],
                            a_hbm.at[row_base + lr])

        @pl.when(s == 0)
        def _():
            @pl.loop(0, _ROWS_PP // _DF)
            def _(k):
                pltpu.sync_copy(
                    spmem.at[pl.ds(_PANEL + k * _DF, _DF)],
                    deg_hbm.at[(row_base // _DF) + k])

        plsc.subcore_barrier()


def _build_adj(edge_index):
    mesh = plsc.VectorSubcoreMesh(core_axis_name="c", subcore_axis_name="s")
    kern = pl.kernel(
        _adj_body,
        out_type=[
            jax.ShapeDtypeStruct((_N, _N), jnp.float32),
            jax.ShapeDtypeStruct((_N // _DF, _DF), jnp.float32),
        ],
        mesh=mesh,
        scratch_types=[
            pltpu.VMEM((_EPT,), jnp.int32),
            pltpu.VMEM((_EPT,), jnp.int32),
            pltpu.VMEM((_NCHUNK, _CHUNK), jnp.int32),
            pltpu.VMEM((_NCHUNK, _CHUNK), jnp.float32),
            pltpu.VMEM((_NCHUNK, _CHUNK), jnp.int32),
            pltpu.VMEM((_ZBUF,), jnp.float32),
            pltpu.VMEM_SHARED((_PANEL + _ROWS_PP,), jnp.float32),
        ],
    )
    return kern(edge_index)


_RB = 256
_G1 = _N // _RB


def _s1_body(a_ref, x_ref, xb_ref, dgr_ref, dgc_ref,
             pw_ref, pb_ref, gw_ref, gb_ref, sw_ref, sb_ref,
             outx_ref, adj_ref, ent_ref, sa2_ref, cross_ref, sg2_ref,
             s_scr, gram_scr):
    p = pl.program_id(0)
    i = pl.program_id(1)
    f32 = jnp.float32
    bf16 = jnp.bfloat16
    dn = (((0,), (0,)), ((), ()))

    @pl.when((p == 0) & (i == 0))
    def _():
        outx_ref[...] = jnp.zeros_like(outx_ref)
        adj_ref[...] = jnp.zeros_like(adj_ref)
        ent_ref[...] = jnp.zeros_like(ent_ref)
        sa2_ref[...] = jnp.zeros_like(sa2_ref)
        cross_ref[...] = jnp.zeros_like(cross_ref)
        sg2_ref[...] = jnp.zeros_like(sg2_ref)
        gram_scr[...] = jnp.zeros_like(gram_scr)

    a = a_ref[...]

    @pl.when(p == 0)
    def _():
        dinv_row = lax.rsqrt(dgr_ref[...] + 1.0)
        dinv_col = lax.rsqrt(dgc_ref[...] + 1.0)
        sa2_ref[...] += jnp.sum(a * a)

        asc = (a * dinv_row).astype(bf16)
        xf = x_ref[...]
        t = jnp.dot(asc, xf.astype(bf16), preferred_element_type=f32)
        t = t + dinv_col * xb_ref[...]

        s_pre = dinv_col * jnp.dot(t, pw_ref[...], preferred_element_type=f32)
        s_pre = s_pre + pb_ref[...]
        m = jnp.max(s_pre, axis=-1, keepdims=True)
        e = jnp.exp(s_pre - m)
        z = jnp.sum(e, axis=-1, keepdims=True)
        s_soft = e / z
        sb16 = s_soft.astype(bf16)
        s_scr[pl.ds(i * _RB, _RB), :] = sb16

        ent_ref[...] += jnp.sum(m + jnp.log(z)) - jnp.sum(s_soft * s_pre)

        xg = dinv_col * jnp.dot(t, gw_ref[...], preferred_element_type=f32)
        x1 = jax.nn.relu(
            xg + gb_ref[...]
            + jnp.dot(xb_ref[...], sw_ref[...], preferred_element_type=f32)
            + sb_ref[...])

        outx_ref[...] += lax.dot_general(
            sb16, x1.astype(bf16), dn, preferred_element_type=f32)
        gram_scr[...] += lax.dot_general(
            sb16, sb16, dn, preferred_element_type=f32)

    @pl.when(p == 1)
    def _():
        sfull = s_scr[...]
        b = jnp.dot(a.astype(bf16), sfull, preferred_element_type=f32)
        sblk = s_scr[pl.ds(i * _RB, _RB), :]
        adj_ref[...] += lax.dot_general(
            sblk, b.astype(bf16), dn, preferred_element_type=f32)
        cross_ref[...] += jnp.sum(sblk.astype(f32) * b)

        @pl.when(i == _G1 - 1)
        def _():
            g = gram_scr[...]
            sg2_ref[...] += jnp.sum(g * g)


def _stage1(a, x, deg_row, deg_col, pw, pb, gw, gb, sw, sb):
    return pl.pallas_call(
        _s1_body,
        grid=(2, _G1),
        in_specs=[
            pl.BlockSpec((_RB, _N), lambda p, i: (i, 0)),
            pl.BlockSpec((_N, _DF), lambda p, i: (0, 0)),
            pl.BlockSpec((_RB, _DF), lambda p, i: (i, 0)),
            pl.BlockSpec((1, _N), lambda p, i: (0, 0)),
            pl.BlockSpec((_RB, 1), lambda p, i: (i, 0)),
            pl.BlockSpec((_DF, _P1), lambda p, i: (0, 0)),
            pl.BlockSpec((1, _P1), lambda p, i: (0, 0)),
            pl.BlockSpec((_DF, _H), lambda p, i: (0, 0)),
            pl.BlockSpec((1, _H), lambda p, i: (0, 0)),
            pl.BlockSpec((_DF, _H), lambda p, i: (0, 0)),
            pl.BlockSpec((1, _H), lambda p, i: (0, 0)),
        ],
        out_specs=[
            pl.BlockSpec((_P1, _H), lambda p, i: (0, 0)),
            pl.BlockSpec((_P1, _P1), lambda p, i: (0, 0)),
            pl.BlockSpec((1, 1), lambda p, i: (0, 0)),
            pl.BlockSpec((1, 1), lambda p, i: (0, 0)),
            pl.BlockSpec((1, 1), lambda p, i: (0, 0)),
            pl.BlockSpec((1, 1), lambda p, i: (0, 0)),
        ],
        out_shape=[
            jax.ShapeDtypeStruct((_P1, _H), jnp.float32),
            jax.ShapeDtypeStruct((_P1, _P1), jnp.float32),
            jax.ShapeDtypeStruct((1, 1), jnp.float32),
            jax.ShapeDtypeStruct((1, 1), jnp.float32),
            jax.ShapeDtypeStruct((1, 1), jnp.float32),
            jax.ShapeDtypeStruct((1, 1), jnp.float32),
        ],
        scratch_shapes=[
            pltpu.VMEM((_N, _P1), jnp.bfloat16),
            pltpu.VMEM((_P1, _P1), jnp.float32),
        ],
    )(a, x, x, deg_row, deg_col, pw, pb, gw, gb, sw, sb)


def _s2_body(x2_ref, a2_ref, ent1_ref, sa2_ref, cross_ref, sg2_ref,
             pw_ref, pb_ref, gw1_ref, gb1_ref, sw1_ref, sb1_ref,
             gw2_ref, gb2_ref, sw2_ref, sb2_ref, cw_ref, cb_ref,
             out_ref, l1_ref, l2_ref):
    f32 = jnp.float32
    bf16 = jnp.bfloat16
    dn = (((0,), (0,)), ((), ()))

    num1 = sa2_ref[0, 0] - 2.0 * cross_ref[0, 0] + sg2_ref[0, 0]
    l1a = jnp.sqrt(jnp.maximum(num1, 0.0)) / (f32(_N) * f32(_N))
    l2a = ent1_ref[0, 0] / f32(_N)

    x2 = x2_ref[...]
    a2 = a2_ref[...]

    deg2 = jnp.sum(a2, axis=1, keepdims=True) + 1.0
    dinv2 = lax.rsqrt(deg2)
    xd2 = dinv2 * x2
    t2 = jnp.dot(a2, xd2, preferred_element_type=f32) + xd2

    s2p = dinv2 * jnp.dot(t2, pw_ref[...], preferred_element_type=f32)
    s2p = s2p + pb_ref[...]
    m = jnp.max(s2p, axis=-1, keepdims=True)
    e = jnp.exp(s2p - m)
    z = jnp.sum(e, axis=-1, keepdims=True)
    s2 = e / z
    s2b = s2.astype(bf16)
    l2b = (jnp.sum(m + jnp.log(z)) - jnp.sum(s2 * s2p)) / f32(_P1)

    xg2 = dinv2 * jnp.dot(t2, gw1_ref[...], preferred_element_type=f32)
    x2b = jax.nn.relu(
        xg2 + gb1_ref[...]
        + jnp.dot(x2, sw1_ref[...], preferred_element_type=f32) + sb1_ref[...])

    b2 = jnp.dot(a2.astype(bf16), s2b, preferred_element_type=f32)
    x3 = lax.dot_general(s2b, x2b.astype(bf16), dn,
                         preferred_element_type=f32)
    a3 = lax.dot_general(s2b, b2.astype(bf16), dn,
                         preferred_element_type=f32)

    gram2 = lax.dot_general(s2b, s2b, dn, preferred_element_type=f32)
    num2 = (jnp.sum(a2 * a2) - 2.0 * jnp.sum(s2 * b2)
            + jnp.sum(gram2 * gram2))
    l1b = jnp.sqrt(jnp.maximum(num2, 0.0)) / (f32(_P1) * f32(_P1))

    deg3 = jnp.sum(a3, axis=1, keepdims=True) + 1.0
    dinv3 = lax.rsqrt(deg3)
    xd3 = dinv3 * x3
    t3 = jnp.dot(a3, xd3, preferred_element_type=f32) + xd3
    xg3 = dinv3 * jnp.dot(t3, gw2_ref[...], preferred_element_type=f32)
    x4 = jax.nn.relu(
        xg3 + gb2_ref[...]
        + jnp.dot(x3, sw2_ref[...], preferred_element_type=f32) + sb2_ref[...])

    pooled = jnp.sum(x4, axis=0, keepdims=True) / f32(_P2)
    out_ref[...] = (jnp.dot(pooled, cw_ref[...], preferred_element_type=f32)
                    + cb_ref[...])
    l1_ref[...] = jnp.full((1, 1), 0.0, f32) + (l1a + l1b)
    l2_ref[...] = jnp.full((1, 1), 0.0, f32) + (l2a + l2b)


def _stage2(x2, a2, ent1, sa2, cross, sg2,
            pw1, pb1, gw1, gb1, sw1, sb1, gw2, gb2, sw2, sb2, cw, cb):
    return pl.pallas_call(
        _s2_body,
        out_shape=[
            jax.ShapeDtypeStruct((1, _NCLS), jnp.float32),
            jax.ShapeDtypeStruct((1, 1), jnp.float32),
            jax.ShapeDtypeStruct((1, 1), jnp.float32),
        ],
    )(x2, a2, ent1, sa2, cross, sg2,
      pw1, pb1, gw1, gb1, sw1, sb1, gw2, gb2, sw2, sb2, cw, cb)


def kernel(x, edge_index, gcn_w0, gcn_b0, gcn_w1, gcn_b1, gcn_w2, gcn_b2,
           skip_w0, skip_b0, skip_w1, skip_b1, skip_w2, skip_b2,
           pool_w0, pool_b0, pool_w1, pool_b1, cls_w, cls_b):
    a, deg = _build_adj(edge_index)
    deg_row = deg.reshape(1, _N)
    deg_col = deg.reshape(_N, 1)
    out_x, out_adj, ent1, sa2, cross, sg2 = _stage1(
        a, x, deg_row, deg_col,
        pool_w0, pool_b0.reshape(1, _P1),
        gcn_w0, gcn_b0.reshape(1, _H),
        skip_w0, skip_b0.reshape(1, _H))
    out, l1, l2 = _stage2(
        out_x, out_adj, ent1, sa2, cross, sg2,
        pool_w1, pool_b1.reshape(1, _P2),
        gcn_w1, gcn_b1.reshape(1, _H),
        skip_w1, skip_b1.reshape(1, _H),
        gcn_w2, gcn_b2.reshape(1, _H),
        skip_w2, skip_b2.reshape(1, _H),
        cls_w, cls_b.reshape(1, _NCLS))
    return out, l1[0, 0], l2[0, 0]


# speedup vs baseline: 1.5205x; 1.5205x over previous
import jax
import jax.numpy as jnp
from jax import lax
from jax.experimental import pallas as pl
from jax.experimental.pallas import tpu as pltpu
from jax.experimental.pallas import tpu_sc as plsc

_N = 2048
_E = 65536
_DF = 128
_H = 32
_P1 = 1024
_P2 = 512
_NCLS = 10


_NC = 2
_NS = 16
_LANES = 16
_ROWS_PP = 512
_PASSES = _N // (_NC * _ROWS_PP)
_EPT = _E // _NS
_CHUNK = 128
_NCHUNK = _EPT // _CHUNK
_ZBUF = 8192
_PANEL = _ROWS_PP * _N
_SP_PER_TILE = _PANEL // _NS
_ROWS_PER_TILE = _ROWS_PP // _NS


def _adj_body(edge_hbm, a_hbm, deg_hbm, src_v, dst_v, idx_v, upd_v, didx_v,
              zero_v, spmem, sem):
    c = lax.axis_index("c")
    s = lax.axis_index("s")
    e0 = s * _EPT

    pltpu.sync_copy(edge_hbm.at[0, pl.ds(e0, _EPT)], src_v)
    pltpu.sync_copy(edge_hbm.at[1, pl.ds(e0, _EPT)], dst_v)

    @pl.loop(0, _ZBUF, step=_LANES)
    def _(i):
        zero_v[pl.ds(i, _LANES)] = jnp.zeros((_LANES,), jnp.float32)

    @pl.loop(0, _PASSES)
    def _(p):
        row_base = c * (_PASSES * _ROWS_PP) + p * _ROWS_PP

        @pl.loop(0, _SP_PER_TILE, step=_ZBUF)
        def _(z):
            pltpu.async_copy(zero_v,
                             spmem.at[pl.ds(s * _SP_PER_TILE + z, _ZBUF)], sem)

        @pl.when(s == 0)
        def _():
            pltpu.async_copy(zero_v.at[pl.ds(0, _ROWS_PP)],
                             spmem.at[pl.ds(_PANEL, _ROWS_PP)], sem).wait()

        @pl.loop(0, _SP_PER_TILE, step=_ZBUF)
        def _(z):
            pltpu.make_async_copy(
                zero_v, spmem.at[pl.ds(s * _SP_PER_TILE + z, _ZBUF)], sem
            ).wait()

        @pl.loop(0, _NCHUNK)
        def _(j):
            @pl.loop(0, _CHUNK, step=_LANES)
            def _(k):
                off = j * _CHUNK + k
                src = src_v[pl.ds(off, _LANES)]
                dst = dst_v[pl.ds(off, _LANES)]
                rel = src - row_base
                inb = (rel >= 0) & (rel < _ROWS_PP)
                row = rel & (_ROWS_PP - 1)
                idx_v[j, pl.ds(k, _LANES)] = row * _N + dst
                didx_v[j, pl.ds(k, _LANES)] = _PANEL + row
                upd_v[j, pl.ds(k, _LANES)] = jnp.where(
                    inb, jnp.float32(1.0), jnp.float32(0.0))

        plsc.subcore_barrier()

        @pl.loop(0, _NCHUNK)
        def _(j):
            pltpu.async_copy(upd_v.at[j], spmem.at[idx_v.at[j]], sem,
                             add=True)

        @pl.loop(0, _NCHUNK)
        def _(j):
            pltpu.async_copy(upd_v.at[j], spmem.at[didx_v.at[j]], sem,
                             add=True)

        @pl.loop(0, _NCHUNK)
        def _(j):
            pltpu.make_async_copy(upd_v.at[j], spmem.at[idx_v.at[j]],
                                  sem).wait()

        @pl.loop(0, _NCHUNK)
        def _(j):
            pltpu.make_async_copy(upd_v.at[j], spmem.at[didx_v.at[j]],
                                  sem).wait()

        plsc.subcore_barrier()

        @pl.loop(0, _ROWS_PER_TILE)
        def _(r):
            lr = s * _ROWS_PER_TILE + r
            pltpu.async_copy(spmem.at[pl.ds(lr * _N, _N)],
                             a_hbm.at[row_base + lr], sem)

        @pl.when(s == 0)
        def _():
            pltpu.async_copy(spmem.at[pl.ds(_PANEL, _ROWS_PP)],
                             deg_hbm.at[0, pl.ds(row_base, _ROWS_PP)],
                             sem).wait()

        @pl.loop(0, _ROWS_PER_TILE)
        def _(r):
            lr = s * _ROWS_PER_TILE + r
            pltpu.make_async_copy(spmem.at[pl.ds(lr * _N, _N)],
                                  a_hbm.at[row_base + lr], sem).wait()

        plsc.subcore_barrier()


def _build_adj(edge_index):
    mesh = plsc.VectorSubcoreMesh(core_axis_name="c", subcore_axis_name="s")
    kern = pl.kernel(
        _adj_body,
        out_type=[
            jax.ShapeDtypeStruct((_N, _N), jnp.float32),
            jax.ShapeDtypeStruct((8, _N), jnp.float32),
        ],
        mesh=mesh,
        scratch_types=[
            pltpu.VMEM((_EPT,), jnp.int32),
            pltpu.VMEM((_EPT,), jnp.int32),
            pltpu.VMEM((_NCHUNK, _CHUNK), jnp.int32),
            pltpu.VMEM((_NCHUNK, _CHUNK), jnp.float32),
            pltpu.VMEM((_NCHUNK, _CHUNK), jnp.int32),
            pltpu.VMEM((_ZBUF,), jnp.float32),
            pltpu.VMEM_SHARED((_PANEL + _ROWS_PP,), jnp.float32),
            pltpu.SemaphoreType.DMA,
        ],
    )
    return kern(edge_index)


_RB = 256
_G1 = _N // _RB


def _s1_body(a_ref, x_ref, xb_ref, dgr_ref,
             pw_ref, pb_ref, gw_ref, gb_ref, sw_ref, sb_ref,
             outx_ref, adj_ref, ent_ref, sa2_ref, cross_ref, sg2_ref,
             s_scr, st_scr, abf_scr, gram_scr):
    p = pl.program_id(0)
    i = pl.program_id(1)
    f32 = jnp.float32
    bf16 = jnp.bfloat16

    @pl.when((p == 0) & (i == 0))
    def _():
        outx_ref[...] = jnp.zeros_like(outx_ref)
        adj_ref[...] = jnp.zeros_like(adj_ref)
        ent_ref[...] = jnp.zeros_like(ent_ref)
        sa2_ref[...] = jnp.zeros_like(sa2_ref)
        cross_ref[...] = jnp.zeros_like(cross_ref)
        sg2_ref[...] = jnp.zeros_like(sg2_ref)
        gram_scr[...] = jnp.zeros_like(gram_scr)

    @pl.when(p == 0)
    def _():
        a = a_ref[...]
        dinv_row = lax.rsqrt(dgr_ref[0:1, :] + 1.0)
        dinv_col = lax.rsqrt(lax.transpose(
            dgr_ref[0:1, pl.ds(i * _RB, _RB)], (1, 0)) + 1.0)
        sa2_ref[...] += jnp.sum(a * a)
        abf_scr[pl.ds(i * _RB, _RB), :] = a.astype(bf16)

        asc = (a * dinv_row).astype(bf16)
        t = jnp.dot(asc, x_ref[...].astype(bf16), preferred_element_type=f32)
        t = t + dinv_col * xb_ref[...]

        s_pre = dinv_col * jnp.dot(t, pw_ref[...], preferred_element_type=f32)
        s_pre = s_pre + pb_ref[...]
        m = jnp.max(s_pre, axis=-1, keepdims=True)
        e = jnp.exp(s_pre - m)
        z = jnp.sum(e, axis=-1, keepdims=True)
        s_soft = e / z
        sb16 = s_soft.astype(bf16)
        st16 = lax.transpose(sb16, (1, 0))
        s_scr[pl.ds(i * _RB, _RB), :] = sb16
        st_scr[:, pl.ds(i * _RB, _RB)] = st16

        ent_ref[...] += jnp.sum(m + jnp.log(z)) - jnp.sum(s_soft * s_pre)

        xg = dinv_col * jnp.dot(t, gw_ref[...], preferred_element_type=f32)
        x1 = jax.nn.relu(
            xg + gb_ref[...]
            + jnp.dot(xb_ref[...], sw_ref[...], preferred_element_type=f32)
            + sb_ref[...])

        outx_ref[...] += jnp.dot(st16, x1.astype(bf16),
                                 preferred_element_type=f32)
        gram_scr[...] += jnp.dot(st16, sb16, preferred_element_type=f32)

    @pl.when(p == 1)
    def _():
        abf = abf_scr[pl.ds(i * _RB, _RB), :]
        b = jnp.dot(abf, s_scr[...], preferred_element_type=f32)
        adj_ref[...] += jnp.dot(st_scr[:, pl.ds(i * _RB, _RB)],
                                b.astype(bf16), preferred_element_type=f32)
        cross_ref[...] += jnp.sum(
            s_scr[pl.ds(i * _RB, _RB), :].astype(f32) * b)

        @pl.when(i == _G1 - 1)
        def _():
            g = gram_scr[...]
            sg2_ref[...] += jnp.sum(g * g)


def _stage1(a, x, deg, pw, pb, gw, gb, sw, sb):
    return pl.pallas_call(
        _s1_body,
        grid=(2, _G1),
        in_specs=[
            pl.BlockSpec((_RB, _N), lambda p, i: (i * (1 - p) + 7 * p, 0)),
            pl.BlockSpec((_N, _DF), lambda p, i: (0, 0)),
            pl.BlockSpec((_RB, _DF), lambda p, i: (i * (1 - p) + 7 * p, 0)),
            pl.BlockSpec((8, _N), lambda p, i: (0, 0)),
            pl.BlockSpec((_DF, _P1), lambda p, i: (0, 0)),
            pl.BlockSpec((1, _P1), lambda p, i: (0, 0)),
            pl.BlockSpec((_DF, _H), lambda p, i: (0, 0)),
            pl.BlockSpec((1, _H), lambda p, i: (0, 0)),
            pl.BlockSpec((_DF, _H), lambda p, i: (0, 0)),
            pl.BlockSpec((1, _H), lambda p, i: (0, 0)),
        ],
        out_specs=[
            pl.BlockSpec((_P1, _H), lambda p, i: (0, 0)),
            pl.BlockSpec((_P1, _P1), lambda p, i: (0, 0)),
            pl.BlockSpec((1, 1), lambda p, i: (0, 0)),
            pl.BlockSpec((1, 1), lambda p, i: (0, 0)),
            pl.BlockSpec((1, 1), lambda p, i: (0, 0)),
            pl.BlockSpec((1, 1), lambda p, i: (0, 0)),
        ],
        out_shape=[
            jax.ShapeDtypeStruct((_P1, _H), jnp.float32),
            jax.ShapeDtypeStruct((_P1, _P1), jnp.float32),
            jax.ShapeDtypeStruct((1, 1), jnp.float32),
            jax.ShapeDtypeStruct((1, 1), jnp.float32),
            jax.ShapeDtypeStruct((1, 1), jnp.float32),
            jax.ShapeDtypeStruct((1, 1), jnp.float32),
        ],
        scratch_shapes=[
            pltpu.VMEM((_N, _P1), jnp.bfloat16),
            pltpu.VMEM((_P1, _N), jnp.bfloat16),
            pltpu.VMEM((_N, _N), jnp.bfloat16),
            pltpu.VMEM((_P1, _P1), jnp.float32),
        ],
    )(a, x, x, deg, pw, pb, gw, gb, sw, sb)


def _s2_body(x2_ref, a2_ref, ent1_ref, sa2_ref, cross_ref, sg2_ref,
             pw_ref, pb_ref, gw1_ref, gb1_ref, sw1_ref, sb1_ref,
             gw2_ref, gb2_ref, sw2_ref, sb2_ref, cw_ref, cb_ref,
             out_ref, l1_ref, l2_ref):
    f32 = jnp.float32
    bf16 = jnp.bfloat16
    dn = (((0,), (0,)), ((), ()))

    num1 = sa2_ref[0, 0] - 2.0 * cross_ref[0, 0] + sg2_ref[0, 0]
    l1a = jnp.sqrt(jnp.maximum(num1, 0.0)) / (f32(_N) * f32(_N))
    l2a = ent1_ref[0, 0] / f32(_N)

    x2 = x2_ref[...]
    a2 = a2_ref[...]

    deg2 = jnp.sum(a2, axis=1, keepdims=True) + 1.0
    dinv2 = lax.rsqrt(deg2)
    xd2 = dinv2 * x2
    t2 = jnp.dot(a2, xd2, preferred_element_type=f32) + xd2

    s2p = dinv2 * jnp.dot(t2, pw_ref[...], preferred_element_type=f32)
    s2p = s2p + pb_ref[...]
    m = jnp.max(s2p, axis=-1, keepdims=True)
    e = jnp.exp(s2p - m)
    z = jnp.sum(e, axis=-1, keepdims=True)
    s2 = e / z
    s2b = s2.astype(bf16)
    l2b = (jnp.sum(m + jnp.log(z)) - jnp.sum(s2 * s2p)) / f32(_P1)

    xg2 = dinv2 * jnp.dot(t2, gw1_ref[...], preferred_element_type=f32)
    x2b = jax.nn.relu(
        xg2 + gb1_ref[...]
        + jnp.dot(x2, sw1_ref[...], preferred_element_type=f32) + sb1_ref[...])

    b2 = jnp.dot(a2.astype(bf16), s2b, preferred_element_type=f32)
    x3 = lax.dot_general(s2b, x2b.astype(bf16), dn,
                         preferred_element_type=f32)
    a3 = lax.dot_general(s2b, b2.astype(bf16), dn,
                         preferred_element_type=f32)

    gram2 = lax.dot_general(s2b, s2b, dn, preferred_element_type=f32)
    num2 = (jnp.sum(a2 * a2) - 2.0 * jnp.sum(s2 * b2)
            + jnp.sum(gram2 * gram2))
    l1b = jnp.sqrt(jnp.maximum(num2, 0.0)) / (f32(_P1) * f32(_P1))

    deg3 = jnp.sum(a3, axis=1, keepdims=True) + 1.0
    dinv3 = lax.rsqrt(deg3)
    xd3 = dinv3 * x3
    t3 = jnp.dot(a3, xd3, preferred_element_type=f32) + xd3
    xg3 = dinv3 * jnp.dot(t3, gw2_ref[...], preferred_element_type=f32)
    x4 = jax.nn.relu(
        xg3 + gb2_ref[...]
        + jnp.dot(x3, sw2_ref[...], preferred_element_type=f32) + sb2_ref[...])

    pooled = jnp.sum(x4, axis=0, keepdims=True) / f32(_P2)
    out_ref[...] = (jnp.dot(pooled, cw_ref[...], preferred_element_type=f32)
                    + cb_ref[...])
    l1_ref[...] = jnp.full((1, 1), 0.0, f32) + (l1a + l1b)
    l2_ref[...] = jnp.full((1, 1), 0.0, f32) + (l2a + l2b)


def _stage2(x2, a2, ent1, sa2, cross, sg2,
            pw1, pb1, gw1, gb1, sw1, sb1, gw2, gb2, sw2, sb2, cw, cb):
    return pl.pallas_call(
        _s2_body,
        out_shape=[
            jax.ShapeDtypeStruct((1, _NCLS), jnp.float32),
            jax.ShapeDtypeStruct((1, 1), jnp.float32),
            jax.ShapeDtypeStruct((1, 1), jnp.float32),
        ],
    )(x2, a2, ent1, sa2, cross, sg2,
      pw1, pb1, gw1, gb1, sw1, sb1, gw2, gb2, sw2, sb2, cw, cb)


def kernel(x, edge_index, gcn_w0, gcn_b0, gcn_w1, gcn_b1, gcn_w2, gcn_b2,
           skip_w0, skip_b0, skip_w1, skip_b1, skip_w2, skip_b2,
           pool_w0, pool_b0, pool_w1, pool_b1, cls_w, cls_b):
    a, deg = _build_adj(edge_index)
    out_x, out_adj, ent1, sa2, cross, sg2 = _stage1(
        a, x, deg,
        pool_w0, pool_b0.reshape(1, _P1),
        gcn_w0, gcn_b0.reshape(1, _H),
        skip_w0, skip_b0.reshape(1, _H))
    out, l1, l2 = _stage2(
        out_x, out_adj, ent1, sa2, cross, sg2,
        pool_w1, pool_b1.reshape(1, _P2),
        gcn_w1, gcn_b1.reshape(1, _H),
        skip_w1, skip_b1.reshape(1, _H),
        gcn_w2, gcn_b2.reshape(1, _H),
        skip_w2, skip_b2.reshape(1, _H),
        cls_w, cls_b.reshape(1, _NCLS))
    return out, l1[0, 0], l2[0, 0]


# speedup vs baseline: 1.6244x; 1.0683x over previous
import jax
import jax.numpy as jnp
from jax import lax
from jax.experimental import pallas as pl
from jax.experimental.pallas import tpu as pltpu
from jax.experimental.pallas import tpu_sc as plsc

_N = 2048
_E = 65536
_DF = 128
_H = 32
_P1 = 1024
_P2 = 512
_NCLS = 10


_NC = 2
_NS = 16
_LANES = 16
_ROWS_PP = 512
_PASSES = _N // (_NC * _ROWS_PP)
_RPC = _PASSES * _ROWS_PP
_EPT = _E // _NS
_CHUNK = 128
_NCHUNK = _EPT // _CHUNK
_ZBUF = 8192
_PANEL = _ROWS_PP * _N
_SP_PER_TILE = _PANEL // _NS
_ROWS_PER_TILE = _ROWS_PP // _NS


def _adj_body(edge_hbm, a_hbm, deg_hbm, src_v, dst_v, idx_v, upd_v,
              didx_v, udeg_v, zero_v, spmem, sem):
    c = lax.axis_index("c")
    s = lax.axis_index("s")
    e0 = s * _EPT

    pltpu.sync_copy(edge_hbm.at[0, pl.ds(e0, _EPT)], src_v)
    pltpu.sync_copy(edge_hbm.at[1, pl.ds(e0, _EPT)], dst_v)

    @pl.loop(0, _ZBUF, step=_LANES)
    def _(i):
        zero_v[pl.ds(i, _LANES)] = jnp.zeros((_LANES,), jnp.float32)

    @pl.loop(0, _NCHUNK)
    def _(j):
        @pl.loop(0, _CHUNK, step=_LANES)
        def _(k):
            off = j * _CHUNK + k
            rel = src_v[pl.ds(off, _LANES)] - c * _RPC
            inb = (rel >= 0) & (rel < _RPC)
            didx_v[j, pl.ds(k, _LANES)] = _PANEL + (rel & (_RPC - 1))
            udeg_v[j, pl.ds(k, _LANES)] = jnp.where(
                inb, jnp.float32(1.0), jnp.float32(0.0))

    @pl.when(s == 0)
    def _():
        pltpu.sync_copy(zero_v.at[pl.ds(0, _RPC)],
                        spmem.at[pl.ds(_PANEL, _RPC)])

    @pl.loop(0, _PASSES)
    def _(p):
        row_base = c * _RPC + p * _ROWS_PP

        @pl.loop(0, _SP_PER_TILE, step=_ZBUF)
        def _(z):
            pltpu.async_copy(zero_v,
                             spmem.at[pl.ds(s * _SP_PER_TILE + z, _ZBUF)], sem)

        @pl.loop(0, _SP_PER_TILE, step=_ZBUF)
        def _(z):
            pltpu.make_async_copy(
                zero_v, spmem.at[pl.ds(s * _SP_PER_TILE + z, _ZBUF)], sem
            ).wait()

        @pl.loop(0, _NCHUNK)
        def _(j):
            @pl.loop(0, _CHUNK, step=_LANES)
            def _(k):
                off = j * _CHUNK + k
                src = src_v[pl.ds(off, _LANES)]
                dst = dst_v[pl.ds(off, _LANES)]
                rel = src - row_base
                inb = (rel >= 0) & (rel < _ROWS_PP)
                row = rel & (_ROWS_PP - 1)
                idx_v[j, pl.ds(k, _LANES)] = row * _N + dst
                upd_v[j, pl.ds(k, _LANES)] = jnp.where(
                    inb, jnp.float32(1.0), jnp.float32(0.0))

        plsc.subcore_barrier()

        @pl.loop(0, _NCHUNK)
        def _(j):
            pltpu.async_copy(upd_v.at[j], spmem.at[idx_v.at[j]], sem,
                             add=True)

        @pl.when(p == 0)
        def _():
            @pl.loop(0, _NCHUNK)
            def _(j):
                pltpu.async_copy(udeg_v.at[j], spmem.at[didx_v.at[j]], sem,
                                 add=True)

            @pl.loop(0, _NCHUNK)
            def _(j):
                pltpu.make_async_copy(udeg_v.at[j], spmem.at[didx_v.at[j]],
                                      sem).wait()

        @pl.loop(0, _NCHUNK)
        def _(j):
            pltpu.make_async_copy(upd_v.at[j], spmem.at[idx_v.at[j]],
                                  sem).wait()

        plsc.subcore_barrier()

        @pl.loop(0, _ROWS_PER_TILE)
        def _(r):
            lr = s * _ROWS_PER_TILE + r
            pltpu.async_copy(spmem.at[pl.ds(lr * _N, _N)],
                             a_hbm.at[row_base + lr], sem)

        @pl.loop(0, _ROWS_PER_TILE)
        def _(r):
            lr = s * _ROWS_PER_TILE + r
            pltpu.make_async_copy(spmem.at[pl.ds(lr * _N, _N)],
                                  a_hbm.at[row_base + lr], sem).wait()

        plsc.subcore_barrier()

    @pl.when(s == 0)
    def _():
        pltpu.sync_copy(spmem.at[pl.ds(_PANEL, _RPC)],
                        deg_hbm.at[0, pl.ds(c * _RPC, _RPC)])


def _build_adj(edge_index):
    mesh = plsc.VectorSubcoreMesh(core_axis_name="c", subcore_axis_name="s")
    kern = pl.kernel(
        _adj_body,
        out_type=[
            jax.ShapeDtypeStruct((_N, _N), jnp.float32),
            jax.ShapeDtypeStruct((8, _N), jnp.float32),
        ],
        mesh=mesh,
        scratch_types=[
            pltpu.VMEM((_EPT,), jnp.int32),
            pltpu.VMEM((_EPT,), jnp.int32),
            pltpu.VMEM((_NCHUNK, _CHUNK), jnp.int32),
            pltpu.VMEM((_NCHUNK, _CHUNK), jnp.float32),
            pltpu.VMEM((_NCHUNK, _CHUNK), jnp.int32),
            pltpu.VMEM((_NCHUNK, _CHUNK), jnp.float32),
            pltpu.VMEM((_ZBUF,), jnp.float32),
            pltpu.VMEM_SHARED((_PANEL + _RPC,), jnp.float32),
            pltpu.SemaphoreType.DMA,
        ],
    )
    return kern(edge_index)


_RB = 256
_G1 = _N // _RB


def _tc_body(a_ref, x_ref, xb_ref, dgr_ref,
             pw_ref, pb_ref, gw_ref, gb_ref, sw_ref, sb_ref,
             pw1_ref, pb1_ref, gw1_ref, gb1_ref, sw1_ref, sb1_ref,
             gw2_ref, gb2_ref, sw2_ref, sb2_ref, cw_ref, cb_ref,
             out_ref, l1_ref, l2_ref,
             s_scr, st_scr, abf_scr, gram_scr, outx_scr, adj_scr,
             sa2_scr, ent_scr, cross_scr, sg2_scr):
    p = pl.program_id(0)
    i = pl.program_id(1)
    f32 = jnp.float32
    bf16 = jnp.bfloat16
    dn = (((0,), (0,)), ((), ()))

    @pl.when((p == 0) & (i == 0))
    def _():
        outx_scr[...] = jnp.zeros_like(outx_scr)
        adj_scr[...] = jnp.zeros_like(adj_scr)
        gram_scr[...] = jnp.zeros_like(gram_scr)
        sa2_scr[...] = jnp.zeros_like(sa2_scr)
        ent_scr[...] = jnp.zeros_like(ent_scr)
        cross_scr[...] = jnp.zeros_like(cross_scr)
        sg2_scr[...] = jnp.zeros_like(sg2_scr)

    @pl.when(p == 0)
    def _():
        a = a_ref[...]
        dinv_row = lax.rsqrt(dgr_ref[0:1, :] + 1.0)
        dinv_col = lax.rsqrt(lax.transpose(
            dgr_ref[0:1, pl.ds(i * _RB, _RB)], (1, 0)) + 1.0)
        sa2_scr[...] += jnp.sum(a * a)
        abf_scr[pl.ds(i * _RB, _RB), :] = a.astype(bf16)

        asc = (a * dinv_row).astype(bf16)
        t = jnp.dot(asc, x_ref[...].astype(bf16), preferred_element_type=f32)
        t = t + dinv_col * xb_ref[...]

        s_pre = dinv_col * jnp.dot(t, pw_ref[...], preferred_element_type=f32)
        s_pre = s_pre + pb_ref[...]
        m = jnp.max(s_pre, axis=-1, keepdims=True)
        e = jnp.exp(s_pre - m)
        z = jnp.sum(e, axis=-1, keepdims=True)
        s_soft = e / z
        sb16 = s_soft.astype(bf16)
        st16 = lax.transpose(sb16, (1, 0))
        s_scr[pl.ds(i * _RB, _RB), :] = sb16
        st_scr[:, pl.ds(i * _RB, _RB)] = st16

        ent_scr[...] += jnp.sum(m + jnp.log(z)) - jnp.sum(s_soft * s_pre)

        xg = dinv_col * jnp.dot(t, gw_ref[...], preferred_element_type=f32)
        x1 = jax.nn.relu(
            xg + gb_ref[...]
            + jnp.dot(xb_ref[...], sw_ref[...], preferred_element_type=f32)
            + sb_ref[...])

        outx_scr[...] += jnp.dot(st16, x1.astype(bf16),
                                 preferred_element_type=f32)
        gram_scr[...] += jnp.dot(st16, sb16, preferred_element_type=f32)

    @pl.when(p == 1)
    def _():
        abf = abf_scr[pl.ds(i * _RB, _RB), :]
        b = jnp.dot(abf, s_scr[...], preferred_element_type=f32)
        adj_scr[...] += jnp.dot(st_scr[:, pl.ds(i * _RB, _RB)],
                                b.astype(bf16), preferred_element_type=f32)
        cross_scr[...] += jnp.sum(
            s_scr[pl.ds(i * _RB, _RB), :].astype(f32) * b)

        @pl.when(i == _G1 - 1)
        def _():
            g = gram_scr[...]
            sg2_scr[...] += jnp.sum(g * g)

    @pl.when((p == 2) & (i == 0))
    def _():
        num1 = sa2_scr[0, 0] - 2.0 * cross_scr[0, 0] + sg2_scr[0, 0]
        l1a = jnp.sqrt(jnp.maximum(num1, 0.0)) / (f32(_N) * f32(_N))
        l2a = ent_scr[0, 0] / f32(_N)

        x2 = outx_scr[...]
        a2 = adj_scr[...]

        deg2 = jnp.sum(a2, axis=1, keepdims=True) + 1.0
        dinv2 = lax.rsqrt(deg2)
        xd2 = dinv2 * x2
        t2 = jnp.dot(a2, xd2, preferred_element_type=f32) + xd2

        s2p = dinv2 * jnp.dot(t2, pw1_ref[...], preferred_element_type=f32)
        s2p = s2p + pb1_ref[...]
        m = jnp.max(s2p, axis=-1, keepdims=True)
        e = jnp.exp(s2p - m)
        z = jnp.sum(e, axis=-1, keepdims=True)
        s2 = e / z
        s2b = s2.astype(bf16)
        l2b = (jnp.sum(m + jnp.log(z)) - jnp.sum(s2 * s2p)) / f32(_P1)

        xg2 = dinv2 * jnp.dot(t2, gw1_ref[...], preferred_element_type=f32)
        x2b = jax.nn.relu(
            xg2 + gb1_ref[...]
            + jnp.dot(x2, sw1_ref[...], preferred_element_type=f32)
            + sb1_ref[...])

        b2 = jnp.dot(a2.astype(bf16), s2b, preferred_element_type=f32)
        x3 = lax.dot_general(s2b, x2b.astype(bf16), dn,
                             preferred_element_type=f32)
        a3 = lax.dot_general(s2b, b2.astype(bf16), dn,
                             preferred_element_type=f32)

        gram2 = lax.dot_general(s2b, s2b, dn, preferred_element_type=f32)
        num2 = (jnp.sum(a2 * a2) - 2.0 * jnp.sum(s2 * b2)
                + jnp.sum(gram2 * gram2))
        l1b = jnp.sqrt(jnp.maximum(num2, 0.0)) / (f32(_P1) * f32(_P1))

        deg3 = jnp.sum(a3, axis=1, keepdims=True) + 1.0
        dinv3 = lax.rsqrt(deg3)
        xd3 = dinv3 * x3
        t3 = jnp.dot(a3, xd3, preferred_element_type=f32) + xd3
        xg3 = dinv3 * jnp.dot(t3, gw2_ref[...], preferred_element_type=f32)
        x4 = jax.nn.relu(
            xg3 + gb2_ref[...]
            + jnp.dot(x3, sw2_ref[...], preferred_element_type=f32)
            + sb2_ref[...])

        pooled = jnp.sum(x4, axis=0, keepdims=True) / f32(_P2)
        out_ref[...] = (jnp.dot(pooled, cw_ref[...],
                                preferred_element_type=f32) + cb_ref[...])
        l1_ref[...] = jnp.full((1, 1), 0.0, f32) + (l1a + l1b)
        l2_ref[...] = jnp.full((1, 1), 0.0, f32) + (l2a + l2b)


def _pipeline(a, x, deg, pw, pb, gw, gb, sw, sb,
              pw1, pb1, gw1, gb1, sw1, sb1, gw2, gb2, sw2, sb2, cw, cb):
    fixed = [
        pl.BlockSpec((_N, _DF), lambda p, i: (0, 0)),
        None,
        pl.BlockSpec((8, _N), lambda p, i: (0, 0)),
        pl.BlockSpec((_DF, _P1), lambda p, i: (0, 0)),
        pl.BlockSpec((1, _P1), lambda p, i: (0, 0)),
        pl.BlockSpec((_DF, _H), lambda p, i: (0, 0)),
        pl.BlockSpec((1, _H), lambda p, i: (0, 0)),
        pl.BlockSpec((_DF, _H), lambda p, i: (0, 0)),
        pl.BlockSpec((1, _H), lambda p, i: (0, 0)),
        pl.BlockSpec((_H, _P2), lambda p, i: (0, 0)),
        pl.BlockSpec((1, _P2), lambda p, i: (0, 0)),
        pl.BlockSpec((_H, _H), lambda p, i: (0, 0)),
        pl.BlockSpec((1, _H), lambda p, i: (0, 0)),
        pl.BlockSpec((_H, _H), lambda p, i: (0, 0)),
        pl.BlockSpec((1, _H), lambda p, i: (0, 0)),
        pl.BlockSpec((_H, _H), lambda p, i: (0, 0)),
        pl.BlockSpec((1, _H), lambda p, i: (0, 0)),
        pl.BlockSpec((_H, _H), lambda p, i: (0, 0)),
        pl.BlockSpec((1, _H), lambda p, i: (0, 0)),
        pl.BlockSpec((_H, _NCLS), lambda p, i: (0, 0)),
        pl.BlockSpec((1, _NCLS), lambda p, i: (0, 0)),
    ]

    def _amap(p, i):
        return (jnp.where(p == 0, i, 7), 0)

    in_specs = [pl.BlockSpec((_RB, _N), _amap)] + fixed
    in_specs[2] = pl.BlockSpec((_RB, _DF), _amap)

    return pl.pallas_call(
        _tc_body,
        grid=(3, _G1),
        in_specs=in_specs,
        out_specs=[
            pl.BlockSpec((1, _NCLS), lambda p, i: (0, 0)),
            pl.BlockSpec((1, 1), lambda p, i: (0, 0)),
            pl.BlockSpec((1, 1), lambda p, i: (0, 0)),
        ],
        out_shape=[
            jax.ShapeDtypeStruct((1, _NCLS), jnp.float32),
            jax.ShapeDtypeStruct((1, 1), jnp.float32),
            jax.ShapeDtypeStruct((1, 1), jnp.float32),
        ],
        scratch_shapes=[
            pltpu.VMEM((_N, _P1), jnp.bfloat16),
            pltpu.VMEM((_P1, _N), jnp.bfloat16),
            pltpu.VMEM((_N, _N), jnp.bfloat16),
            pltpu.VMEM((_P1, _P1), jnp.float32),
            pltpu.VMEM((_P1, _H), jnp.float32),
            pltpu.VMEM((_P1, _P1), jnp.float32),
            pltpu.VMEM((1, 1), jnp.float32),
            pltpu.VMEM((1, 1), jnp.float32),
            pltpu.VMEM((1, 1), jnp.float32),
            pltpu.VMEM((1, 1), jnp.float32),
        ],
    )(a, x, x, deg, pw, pb, gw, gb, sw, sb,
      pw1, pb1, gw1, gb1, sw1, sb1, gw2, gb2, sw2, sb2, cw, cb)


def kernel(x, edge_index, gcn_w0, gcn_b0, gcn_w1, gcn_b1, gcn_w2, gcn_b2,
           skip_w0, skip_b0, skip_w1, skip_b1, skip_w2, skip_b2,
           pool_w0, pool_b0, pool_w1, pool_b1, cls_w, cls_b):
    a, deg = _build_adj(edge_index)
    out, l1, l2 = _pipeline(
        a, x, deg,
        pool_w0, pool_b0.reshape(1, _P1),
        gcn_w0, gcn_b0.reshape(1, _H),
        skip_w0, skip_b0.reshape(1, _H),
        pool_w1, pool_b1.reshape(1, _P2),
        gcn_w1, gcn_b1.reshape(1, _H),
        skip_w1, skip_b1.reshape(1, _H),
        gcn_w2, gcn_b2.reshape(1, _H),
        skip_w2, skip_b2.reshape(1, _H),
        cls_w, cls_b.reshape(1, _NCLS))
    return out, l1[0, 0], l2[0, 0]
